# baseline (device time: 104646 ns/iter reference)
import functools

import jax
import jax.numpy as jnp
from jax import lax
from jax.experimental import pallas as pl
from jax.experimental.pallas import tpu as pltpu

N_DEV = 4


def kernel(A, B):
    m, k = A.shape
    _, n = B.shape
    hm = m // 2
    qm = m // 4
    half = n // 2

    f32 = jnp.float32
    bf16 = jnp.bfloat16

    def body(a_ref, b_ref, out_ref, sb1, rb1, sb2, rb2, sb3, rb3, rb4,
             s1s, s1r, s2s, s2r, s3s, s3r, s4s, s4r):
        my = lax.axis_index("i")
        x = my // 2
        y = (my % 2) ^ x
        px = 3 - my
        py = my ^ 1
        left = (my + N_DEV - 1) % N_DEV
        right = (my + 1) % N_DEV

        barrier_sem = pltpu.get_barrier_semaphore()
        for nbr in (left, right):
            pl.semaphore_signal(
                barrier_sem, inc=1,
                device_id=(nbr,), device_id_type=pl.DeviceIdType.MESH,
            )
        pl.semaphore_wait(barrier_sem, 2)

        cols = (slice(0, half), slice(half, n))
        maj = (x, y)
        mnr = (y, x)
        partner1 = (px, py)
        partner2 = (py, px)
        own_h = tuple(c * hm for c in maj)
        oth_h = tuple((1 - c) * hm for c in maj)
        own_q = tuple(own_h[h] + mnr[h] * qm for h in range(2))
        oth_q = tuple(own_h[h] + (1 - mnr[h]) * qm for h in range(2))
        r4a = tuple(oth_h[h] + mnr[h] * qm for h in range(2))
        r4b = tuple(oth_h[h] + (1 - mnr[h]) * qm for h in range(2))

        def mesh_copy(src, dst, ssem, rsem, dev):
            return pltpu.make_async_remote_copy(
                src_ref=src, dst_ref=dst, send_sem=ssem, recv_sem=rsem,
                device_id=(dev,), device_id_type=pl.DeviceIdType.MESH,
            )

        st1 = [[mesh_copy(sb1.at[h, pl.ds(u * qm, qm), :],
                          rb1.at[h, pl.ds(u * qm, qm), :],
                          s1s.at[h, u], s1r.at[h, u], partner1[h])
                for u in range(2)] for h in range(2)]
        st2 = [mesh_copy(sb2.at[h], rb2.at[h], s2s.at[h], s2r.at[h],
                         partner2[h]) for h in range(2)]
        st3 = [mesh_copy(sb3.at[h], rb3.at[h], s3s.at[h], s3r.at[h],
                         partner2[h]) for h in range(2)]
        st4 = [[mesh_copy((sb3 if u == 0 else rb3).at[h],
                          rb4.at[h, pl.ds(u * qm, qm), :],
                          s4s.at[h, u], s4r.at[h, u], partner1[h])
                for u in range(2)] for h in range(2)]

        def half_dot(row_start, h):
            return jnp.dot(
                a_ref[pl.ds(row_start, qm), :], b_ref[:, cols[h]],
                preferred_element_type=f32,
            )

        for u in range(2):
            for h in range(2):
                sb1[h, pl.ds(u * qm, qm), :] = (
                    half_dot(oth_h[h] + u * qm, h).astype(bf16)
                )
                st1[h][u].start()

        for h in range(2):
            for u in range(2):
                out_ref[pl.ds(own_h[h] + u * qm, qm), cols[h]] = (
                    half_dot(own_h[h] + u * qm, h)
                )

        for h in range(2):
            for u in range(2):
                st1[h][u].wait_recv()
                rows = pl.ds(own_h[h] + u * qm, qm)
                out_ref[rows, cols[h]] = (
                    out_ref[rows, cols[h]]
                    + rb1[h, pl.ds(u * qm, qm), :].astype(f32)
                )
            sb2[h] = out_ref[pl.ds(oth_q[h], qm), cols[h]].astype(bf16)
            st2[h].start()

        for h in range(2):
            st2[h].wait_recv()
            rows = pl.ds(own_q[h], qm)
            acc = out_ref[rows, cols[h]] + rb2[h].astype(f32)
            out_ref[rows, cols[h]] = acc
            sb3[h] = acc.astype(bf16)
            st3[h].start()
            st4[h][0].start()

        for h in range(2):
            st3[h].wait_recv()
            st4[h][1].start()
            out_ref[pl.ds(oth_q[h], qm), cols[h]] = rb3[h].astype(f32)

        for h in range(2):
            for u in range(2):
                st4[h][u].wait_recv()
            out_ref[pl.ds(r4a[h], qm), cols[h]] = (
                rb4[h, pl.ds(0, qm), :].astype(f32)
            )
            out_ref[pl.ds(r4b[h], qm), cols[h]] = (
                rb4[h, pl.ds(qm, qm), :].astype(f32)
            )

        for h in range(2):
            for u in range(2):
                st1[h][u].wait_send()
                st4[h][u].wait_send()
            st2[h].wait_send()
            st3[h].wait_send()

        @functools.partial(
            pl.run_scoped, second_barrier=pltpu.SemaphoreType.REGULAR
        )
        def _(second_barrier):
            for nbr in (left, right):
                pl.semaphore_signal(
                    second_barrier, inc=1,
                    device_id=(nbr,), device_id_type=pl.DeviceIdType.MESH,
                )
            pl.semaphore_wait(second_barrier, 2)

    return pl.pallas_call(
        body,
        out_shape=jax.ShapeDtypeStruct((m, n), f32),
        in_specs=[
            pl.BlockSpec(memory_space=pltpu.VMEM),
            pl.BlockSpec(memory_space=pltpu.VMEM),
        ],
        out_specs=pl.BlockSpec(memory_space=pltpu.VMEM),
        scratch_shapes=[
            pltpu.VMEM((2, hm, half), bf16),
            pltpu.VMEM((2, hm, half), bf16),
            pltpu.VMEM((2, qm, half), bf16),
            pltpu.VMEM((2, qm, half), bf16),
            pltpu.VMEM((2, qm, half), bf16),
            pltpu.VMEM((2, qm, half), bf16),
            pltpu.VMEM((2, hm, half), bf16),
            pltpu.SemaphoreType.DMA((2, 2)),
            pltpu.SemaphoreType.DMA((2, 2)),
            pltpu.SemaphoreType.DMA((2,)),
            pltpu.SemaphoreType.DMA((2,)),
            pltpu.SemaphoreType.DMA((2,)),
            pltpu.SemaphoreType.DMA((2,)),
            pltpu.SemaphoreType.DMA((2, 2)),
            pltpu.SemaphoreType.DMA((2, 2)),
        ],
        compiler_params=pltpu.CompilerParams(
            collective_id=0, vmem_limit_bytes=100 * 1024 * 1024
        ),
    )(A, B)
